# baseline (speedup 1.0000x reference)
import functools
import jax
import jax.numpy as jnp
from jax import lax
from jax.experimental import pallas as pl
from jax.experimental.pallas import tpu as pltpu
from jax.experimental.pallas import tpu_sc as plsc

_EMB_D = 768
_BB = 2048
_NW = 32
_RREP = 128
_SHROWS = 512


def _copy_body(x_ref, xc_ref):
    xc_ref[...] = x_ref[...]


def _make_sc_broadcast(B):
    rows_per_w = B // _NW
    mesh = plsc.VectorSubcoreMesh(core_axis_name="c", subcore_axis_name="s")

    @functools.partial(
        pl.kernel,
        mesh=mesh,
        out_type=jax.ShapeDtypeStruct((B, 1, _EMB_D), jnp.float32),
        scratch_types=[
            pltpu.VMEM((8,), jnp.int32),
            pltpu.VMEM((_RREP, 1, _EMB_D), jnp.float32),
            pltpu.VMEM_SHARED((_SHROWS, 1, _EMB_D), jnp.float32),
            pltpu.SemaphoreType.DMA,
        ],
    )
    def sc_broadcast(pool_hbm, idx_hbm, out_hbm, idx_v, rows_v, shared, sem):
        cid = lax.axis_index("c")
        sid = lax.axis_index("s")
        base = cid * (B // 2) + sid * rows_per_w
        pltpu.sync_copy(idx_hbm, idx_v)
        pltpu.async_copy(pool_hbm.at[idx_v], rows_v.at[pl.ds(0, 8)], sem).wait()
        row = [rows_v[0, 0, pl.ds(16 * i, 16)] for i in range(_EMB_D // 16)]

        def _rep(r, carry):
            for i in range(_EMB_D // 16):
                rows_v[r, 0, pl.ds(16 * i, 16)] = row[i]
            return carry

        lax.fori_loop(8, _RREP, _rep, 0)

        @pl.when(sid < _SHROWS // _RREP)
        def _fill():
            pltpu.sync_copy(
                rows_v, shared.at[pl.ds(sid * _RREP, _RREP)])

        plsc.subcore_barrier()
        pltpu.async_copy(
            shared, out_hbm.at[pl.ds(base, rows_per_w)], sem).wait()

    return sc_broadcast


def kernel(x_querry, l, x_block, e_p, task_id):
    B = x_querry.shape[0]
    xc = pl.pallas_call(
        _copy_body,
        grid=(B // _BB,),
        in_specs=[pl.BlockSpec((_BB, _EMB_D), lambda i: (i, 0))],
        out_specs=pl.BlockSpec((_BB, _EMB_D), lambda i: (i, 0)),
        out_shape=jax.ShapeDtypeStruct((B, _EMB_D), jnp.float32),
    )(x_block)
    l_i = jnp.asarray(l, jnp.int32)
    valid = (l_i >= 0) & (l_i < 12)
    pool = jnp.concatenate(
        [e_p, jnp.full((1, 1, _EMB_D), jnp.nan, jnp.float32)], axis=0)
    sel = jnp.where(valid, jnp.asarray(task_id, jnp.int32), e_p.shape[0])
    idx = jnp.full((8,), sel, jnp.int32)
    P = _make_sc_broadcast(B)(pool, idx)
    return (P, xc)


# speedup vs baseline: 1.6176x; 1.6176x over previous
import jax
import jax.numpy as jnp
from jax.experimental import pallas as pl
from jax.experimental.pallas import tpu as pltpu

_EMB_D = 768
_BB = 2048


def _body(scalars_ref, pool_ref, x_ref, p_ref, xc_ref):
    @pl.when(pl.program_id(0) < 2)
    def _fill():
        tid = scalars_ref[0]
        valid = scalars_ref[1]
        row = pool_ref[pl.ds(tid, 1), :]
        row = jnp.where(valid == 1, row, jnp.full_like(row, jnp.nan))
        p_ref[...] = jnp.broadcast_to(row, p_ref.shape)

    xc_ref[...] = x_ref[...]


def kernel(x_querry, l, x_block, e_p, task_id):
    B = x_querry.shape[0]
    pool = e_p.reshape(e_p.shape[0] * e_p.shape[1], _EMB_D)
    l_i = jnp.asarray(l, jnp.int32)
    valid = ((l_i >= 0) & (l_i < 12)).astype(jnp.int32)
    scalars = jnp.stack([jnp.asarray(task_id, jnp.int32), valid])
    P, xc = pl.pallas_call(
        _body,
        grid_spec=pltpu.PrefetchScalarGridSpec(
            num_scalar_prefetch=1,
            grid=(B // _BB,),
            in_specs=[
                pl.BlockSpec((pool.shape[0], _EMB_D), lambda i, s: (0, 0)),
                pl.BlockSpec((_BB, _EMB_D), lambda i, s: (i, 0)),
            ],
            out_specs=[
                pl.BlockSpec((_BB, None, _EMB_D), lambda i, s: (i, 0, 0)),
                pl.BlockSpec((_BB, _EMB_D), lambda i, s: (i, 0)),
            ],
        ),
        out_shape=[
            jax.ShapeDtypeStruct((B, e_p.shape[1], _EMB_D), jnp.float32),
            jax.ShapeDtypeStruct((B, _EMB_D), jnp.float32),
        ],
    )(scalars, pool, x_block)
    return (P, xc)


# speedup vs baseline: 1.6610x; 1.0268x over previous
import jax
import jax.numpy as jnp
from jax.experimental import pallas as pl
from jax.experimental.pallas import tpu as pltpu

_EMB_D = 768
_BB = 3328


def _body(scalars_ref, pool_ref, x_ref, p_ref, xc_ref):
    tid = scalars_ref[0]
    valid = scalars_ref[1]
    row = pool_ref[pl.ds(tid, 1), :]
    row = jnp.where(valid == 1, row, jnp.full_like(row, jnp.nan))
    p_ref[...] = jnp.broadcast_to(row, p_ref.shape)
    xc_ref[...] = x_ref[...]


def kernel(x_querry, l, x_block, e_p, task_id):
    B = x_querry.shape[0]
    pool = e_p.reshape(e_p.shape[0] * e_p.shape[1], _EMB_D)
    l_i = jnp.asarray(l, jnp.int32)
    valid = ((l_i >= 0) & (l_i < 12)).astype(jnp.int32)
    scalars = jnp.stack([jnp.asarray(task_id, jnp.int32), valid])
    P, xc = pl.pallas_call(
        _body,
        grid_spec=pltpu.PrefetchScalarGridSpec(
            num_scalar_prefetch=1,
            grid=(pl.cdiv(B, _BB),),
            in_specs=[
                pl.BlockSpec((pool.shape[0], _EMB_D), lambda i, s: (0, 0)),
                pl.BlockSpec((_BB, _EMB_D), lambda i, s: (i, 0)),
            ],
            out_specs=[
                pl.BlockSpec((_BB, None, _EMB_D), lambda i, s: (i, 0, 0)),
                pl.BlockSpec((_BB, _EMB_D), lambda i, s: (i, 0)),
            ],
        ),
        out_shape=[
            jax.ShapeDtypeStruct((B, e_p.shape[1], _EMB_D), jnp.float32),
            jax.ShapeDtypeStruct((B, _EMB_D), jnp.float32),
        ],
    )(scalars, pool, x_block)
    return (P, xc)
